# speedup vs baseline: 7.3523x; 7.3523x over previous
import functools

import jax
import jax.numpy as jnp
from jax import lax
from jax.experimental import pallas as pl
from jax.experimental.pallas import tpu as pltpu
from jax.experimental.pallas import tpu_sc as plsc

N_NODES = 10000
N_EDGES = 320000
D = 128
N_GRAPHS = 64

NC, NS = 2, 16
NW = NC * NS
CHUNK = 128
NPAD = 10240
EPAD = 327680
EW = EPAD // NW
ROWS_W = NPAD // NS
PAD_ROW = N_NODES + 8

_SC_MESH = plsc.VectorSubcoreMesh(core_axis_name="c", subcore_axis_name="s")


def _deg_body(dst_hbm, out_hbm, dst_v, ones_v, zero_v, acc_sh):
    c = lax.axis_index("c")
    s = lax.axis_index("s")
    wid = s * NC + c

    @pl.loop(0, CHUNK)
    def _(i):
        ones_v[i, :] = jnp.ones((16,), jnp.float32)
        zero_v[i, :] = jnp.zeros((16,), jnp.float32)

    @pl.loop(0, ROWS_W, step=CHUNK)
    def _(j):
        pltpu.sync_copy(zero_v, acc_sh.at[pl.ds(s * ROWS_W + j, CHUNK)])

    plsc.subcore_barrier()

    @pl.loop(0, EW, step=CHUNK)
    def _(k):
        pltpu.sync_copy(dst_hbm.at[pl.ds(wid * EW + k, CHUNK)], dst_v)
        pltpu.sync_copy(ones_v, acc_sh.at[dst_v], add=True)

    plsc.subcore_barrier()
    pltpu.sync_copy(acc_sh.at[pl.ds(s * ROWS_W, ROWS_W)],
                    out_hbm.at[c].at[pl.ds(s * ROWS_W, ROWS_W)])


@jax.jit
def _sc_degree(dst_p):
    kern = pl.kernel(
        _deg_body,
        out_type=jax.ShapeDtypeStruct((NC, NPAD, 16), jnp.float32),
        mesh=_SC_MESH,
        scratch_types=[
            pltpu.VMEM((CHUNK,), jnp.int32),
            pltpu.VMEM((CHUNK, 16), jnp.float32),
            pltpu.VMEM((CHUNK, 16), jnp.float32),
            pltpu.VMEM_SHARED((NPAD, 16), jnp.float32),
        ],
    )
    return kern(dst_p)


def _agg_body(xs_hbm, src_hbm, dst_hbm, out_hbm, src_v, dst_v, rows_v, acc_sh):
    c = lax.axis_index("c")
    s = lax.axis_index("s")
    wid = s * NC + c

    @pl.loop(0, CHUNK)
    def _(i):
        @pl.loop(0, D, step=16)
        def _(j):
            rows_v[i, pl.ds(j, 16)] = jnp.zeros((16,), jnp.float32)

    @pl.loop(0, ROWS_W, step=CHUNK)
    def _(j):
        pltpu.sync_copy(rows_v, acc_sh.at[pl.ds(s * ROWS_W + j, CHUNK)])

    plsc.subcore_barrier()

    @pl.loop(0, EW, step=CHUNK)
    def _(k):
        base = wid * EW + k
        pltpu.sync_copy(src_hbm.at[pl.ds(base, CHUNK)], src_v)
        pltpu.sync_copy(dst_hbm.at[pl.ds(base, CHUNK)], dst_v)
        pltpu.sync_copy(xs_hbm.at[src_v], rows_v)
        pltpu.sync_copy(rows_v, acc_sh.at[dst_v], add=True)

    plsc.subcore_barrier()
    pltpu.sync_copy(acc_sh.at[pl.ds(s * ROWS_W, ROWS_W)],
                    out_hbm.at[c].at[pl.ds(s * ROWS_W, ROWS_W)])


@jax.jit
def _sc_aggregate(xs, src_p, dst_p):
    kern = pl.kernel(
        _agg_body,
        out_type=jax.ShapeDtypeStruct((NC, NPAD, D), jnp.float32),
        mesh=_SC_MESH,
        scratch_types=[
            pltpu.VMEM((CHUNK,), jnp.int32),
            pltpu.VMEM((CHUNK,), jnp.int32),
            pltpu.VMEM((CHUNK, D), jnp.float32),
            pltpu.VMEM_SHARED((NPAD, D), jnp.float32),
        ],
    )
    return kern(xs, src_p, dst_p)


_BLK = 1280
_NBLK = NPAD // _BLK


def _dinv_of(dega_blk, degb_blk):
    deg = dega_blk[:, 0:1] + degb_blk[:, 0:1] + 1.0
    return lax.rsqrt(deg)


def _tc1_body(x_ref, w_ref, dega_ref, degb_ref, xs_ref):
    xw = jnp.dot(x_ref[...], w_ref[...], preferred_element_type=jnp.float32)
    xs_ref[...] = _dinv_of(dega_ref[...], degb_ref[...]) * xw


@jax.jit
def _tc_xs1(x_p, W1, dega, degb):
    return pl.pallas_call(
        _tc1_body,
        grid=(_NBLK,),
        in_specs=[
            pl.BlockSpec((_BLK, D), lambda i: (i, 0)),
            pl.BlockSpec((D, D), lambda i: (0, 0)),
            pl.BlockSpec((_BLK, 16), lambda i: (i, 0)),
            pl.BlockSpec((_BLK, 16), lambda i: (i, 0)),
        ],
        out_specs=pl.BlockSpec((_BLK, D), lambda i: (i, 0)),
        out_shape=jax.ShapeDtypeStruct((NPAD, D), jnp.float32),
    )(x_p, W1, dega, degb)


def _tc2_body(agga_ref, aggb_ref, xs_ref, dega_ref, degb_ref, b_ref, w_ref,
              out_ref):
    dinv = _dinv_of(dega_ref[...], degb_ref[...])
    pre = dinv * (agga_ref[...] + aggb_ref[...] + xs_ref[...]) + b_ref[...]
    h = jnp.maximum(pre, 0.0)
    hw = jnp.dot(h, w_ref[...], preferred_element_type=jnp.float32)
    out_ref[...] = dinv * hw


@jax.jit
def _tc_xs2(agga, aggb, xs1, dega, degb, b1, W2):
    return pl.pallas_call(
        _tc2_body,
        grid=(_NBLK,),
        in_specs=[
            pl.BlockSpec((_BLK, D), lambda i: (i, 0)),
            pl.BlockSpec((_BLK, D), lambda i: (i, 0)),
            pl.BlockSpec((_BLK, D), lambda i: (i, 0)),
            pl.BlockSpec((_BLK, 16), lambda i: (i, 0)),
            pl.BlockSpec((_BLK, 16), lambda i: (i, 0)),
            pl.BlockSpec((1, D), lambda i: (0, 0)),
            pl.BlockSpec((D, D), lambda i: (0, 0)),
        ],
        out_specs=pl.BlockSpec((_BLK, D), lambda i: (i, 0)),
        out_shape=jax.ShapeDtypeStruct((NPAD, D), jnp.float32),
    )(agga, aggb, xs1, dega, degb, b1, W2)


def _tc3_body(agga_ref, aggb_ref, xs_ref, dega_ref, degb_ref, b_ref,
              batch_ref, wfc_ref, bfc_ref, out_ref, sums_ref, counts_ref):
    i = pl.program_id(0)

    @pl.when(i == 0)
    def _():
        sums_ref[...] = jnp.zeros_like(sums_ref)
        counts_ref[...] = jnp.zeros_like(counts_ref)

    dinv = _dinv_of(dega_ref[...], degb_ref[...])
    pre = dinv * (agga_ref[...] + aggb_ref[...] + xs_ref[...]) + b_ref[...]
    h = jnp.maximum(pre, 0.0)

    gids = lax.broadcasted_iota(jnp.int32, (_BLK, N_GRAPHS), 1)
    onehot = (batch_ref[...] == gids).astype(jnp.float32)
    dn = (((0,), (0,)), ((), ()))
    sums_ref[...] += lax.dot_general(onehot, h, dn,
                                     preferred_element_type=jnp.float32)
    counts_ref[...] += lax.dot_general(
        onehot, jnp.ones((_BLK, D), jnp.float32), dn,
        preferred_element_type=jnp.float32)

    @pl.when(i == _NBLK - 1)
    def _():
        pooled = sums_ref[...] / jnp.maximum(counts_ref[...], 1.0)
        o = jnp.dot(pooled, wfc_ref[...], preferred_element_type=jnp.float32)
        out_ref[...] = jnp.maximum(o + bfc_ref[...], 0.0)


@jax.jit
def _tc_pool(agga, aggb, xs2, dega, degb, b2, batch_col, Wfc, bfc):
    return pl.pallas_call(
        _tc3_body,
        grid=(_NBLK,),
        in_specs=[
            pl.BlockSpec((_BLK, D), lambda i: (i, 0)),
            pl.BlockSpec((_BLK, D), lambda i: (i, 0)),
            pl.BlockSpec((_BLK, D), lambda i: (i, 0)),
            pl.BlockSpec((_BLK, 16), lambda i: (i, 0)),
            pl.BlockSpec((_BLK, 16), lambda i: (i, 0)),
            pl.BlockSpec((1, D), lambda i: (0, 0)),
            pl.BlockSpec((_BLK, 1), lambda i: (i, 0)),
            pl.BlockSpec((D, D), lambda i: (0, 0)),
            pl.BlockSpec((1, D), lambda i: (0, 0)),
        ],
        out_specs=pl.BlockSpec((N_GRAPHS, D), lambda i: (0, 0)),
        out_shape=jax.ShapeDtypeStruct((N_GRAPHS, D), jnp.float32),
        scratch_shapes=[
            pltpu.VMEM((N_GRAPHS, D), jnp.float32),
            pltpu.VMEM((N_GRAPHS, D), jnp.float32),
        ],
    )(agga, aggb, xs2, dega, degb, b2, batch_col, Wfc, bfc)


def kernel(x, edge_index, batch, W1, b1, W2, b2, Wfc, bfc):
    src = edge_index[0].astype(jnp.int32)
    dst = edge_index[1].astype(jnp.int32)
    fill = jnp.full((EPAD - N_EDGES,), PAD_ROW, jnp.int32)
    src_p = jnp.concatenate([src, fill])
    dst_p = jnp.concatenate([dst, fill])
    x_p = jnp.pad(x, ((0, NPAD - N_NODES), (0, 0)))
    batch_col = jnp.pad(batch.astype(jnp.int32), (0, NPAD - N_NODES),
                        constant_values=N_GRAPHS).reshape(NPAD, 1)
    b1r = b1.reshape(1, D)
    b2r = b2.reshape(1, D)
    bfcr = bfc.reshape(1, D)

    deg = _sc_degree(dst_p)
    dega, degb = deg[0], deg[1]
    xs1 = _tc_xs1(x_p, W1, dega, degb)
    agg1 = _sc_aggregate(xs1, src_p, dst_p)
    xs2 = _tc_xs2(agg1[0], agg1[1], xs1, dega, degb, b1r, W2)
    agg2 = _sc_aggregate(xs2, src_p, dst_p)
    return _tc_pool(agg2[0], agg2[1], xs2, dega, degb, b2r, batch_col, Wfc, bfcr)


# speedup vs baseline: 9.9373x; 1.3516x over previous
import functools

import jax
import jax.numpy as jnp
from jax import lax
from jax.experimental import pallas as pl
from jax.experimental.pallas import tpu as pltpu
from jax.experimental.pallas import tpu_sc as plsc

N_NODES = 10000
N_EDGES = 320000
D = 128
N_GRAPHS = 64

NC, NS = 2, 16
NW = NC * NS
CHUNK = 128
NPAD = 10240
EPAD = 327680
EW = EPAD // NW
ROWS_W = NPAD // NS
PAD_ROW = N_NODES + 8

_SC_MESH = plsc.VectorSubcoreMesh(core_axis_name="c", subcore_axis_name="s")


def _deg_body(dst_hbm, out_hbm, dst_v, ones_v, zero_v, acc_sh):
    c = lax.axis_index("c")
    s = lax.axis_index("s")
    wid = s * NC + c

    @pl.loop(0, CHUNK)
    def _(i):
        ones_v[i, :] = jnp.ones((16,), jnp.float32)
        zero_v[i, :] = jnp.zeros((16,), jnp.float32)

    @pl.loop(0, ROWS_W, step=CHUNK)
    def _(j):
        pltpu.sync_copy(zero_v, acc_sh.at[pl.ds(s * ROWS_W + j, CHUNK)])

    plsc.subcore_barrier()

    @pl.loop(0, EW, step=CHUNK)
    def _(k):
        pltpu.sync_copy(dst_hbm.at[pl.ds(wid * EW + k, CHUNK)], dst_v)
        pltpu.sync_copy(ones_v, acc_sh.at[dst_v], add=True)

    plsc.subcore_barrier()
    pltpu.sync_copy(acc_sh.at[pl.ds(s * ROWS_W, ROWS_W)],
                    out_hbm.at[c].at[pl.ds(s * ROWS_W, ROWS_W)])


@jax.jit
def _sc_degree(dst_p):
    kern = pl.kernel(
        _deg_body,
        out_type=jax.ShapeDtypeStruct((NC, NPAD, 16), jnp.float32),
        mesh=_SC_MESH,
        scratch_types=[
            pltpu.VMEM((CHUNK,), jnp.int32),
            pltpu.VMEM((CHUNK, 16), jnp.float32),
            pltpu.VMEM((CHUNK, 16), jnp.float32),
            pltpu.VMEM_SHARED((NPAD, 16), jnp.float32),
        ],
    )
    return kern(dst_p)


NB = 4
NCH = EW // CHUNK


def _agg_body(xs_hbm, ei_hbm, out_hbm, idx_v, rows, isems, gsems, ssems,
              acc_sh):
    c = lax.axis_index("c")
    s = lax.axis_index("s")
    wid = s * NC + c
    gbase = wid * NCH

    r0 = rows[0]

    @pl.loop(0, CHUNK)
    def _(i):
        @pl.loop(0, D, step=16)
        def _(j):
            r0[i, pl.ds(j, 16)] = jnp.zeros((16,), jnp.float32)

    @pl.loop(0, ROWS_W, step=CHUNK)
    def _(j):
        pltpu.sync_copy(r0, acc_sh.at[pl.ds(s * ROWS_W + j, CHUNK)])

    plsc.subcore_barrier()

    def load_idx(ch, q):
        pltpu.async_copy(ei_hbm.at[gbase + ch], idx_v.at[q], isems[q])

    def wait_idx(q):
        pltpu.make_async_copy(ei_hbm.at[gbase], idx_v.at[q], isems[q]).wait()

    def start_gather(q, r):
        pltpu.async_copy(xs_hbm.at[idx_v.at[q].at[0]], rows[r], gsems[r])

    def wait_gather(r):
        pltpu.make_async_copy(xs_hbm.at[idx_v.at[0].at[0]], rows[r],
                              gsems[r]).wait()

    def start_scatter(q, r):
        pltpu.async_copy(rows[r], acc_sh.at[idx_v.at[q].at[1]], ssems[r],
                         add=True)

    def wait_scatter(r):
        pltpu.make_async_copy(rows[r], acc_sh.at[idx_v.at[0].at[1]],
                              ssems[r]).wait()

    load_idx(0, 0)
    load_idx(1, 1)
    load_idx(2, 2)
    wait_idx(0)
    start_gather(0, 0)

    @pl.loop(0, NCH // 4)
    def _(t):
        for j in range(4):
            ch = t * 4 + j
            r, rn = j % 2, (j + 1) % 2

            if j == 0:
                @pl.when(t > 0)
                def _():
                    wait_scatter(rn)
            else:
                wait_scatter(rn)

            def steps23():
                wait_idx((j + 1) % 4)
                start_gather((j + 1) % 4, rn)

            if j < 3:
                steps23()
            else:
                @pl.when(t < NCH // 4 - 1)
                def _():
                    steps23()

            if j == 0:
                load_idx(ch + 3, 3)
            else:
                @pl.when(t < NCH // 4 - 1)
                def _():
                    load_idx(ch + 3, (j + 3) % 4)

            wait_gather(r)
            start_scatter(j, r)

    wait_scatter((NCH - 1) % 2)

    plsc.subcore_barrier()
    pltpu.sync_copy(acc_sh.at[pl.ds(s * ROWS_W, ROWS_W)],
                    out_hbm.at[c].at[pl.ds(s * ROWS_W, ROWS_W)])


@jax.jit
def _sc_aggregate(xs, ei2):
    kern = pl.kernel(
        _agg_body,
        out_type=jax.ShapeDtypeStruct((NC, NPAD, D), jnp.float32),
        mesh=_SC_MESH,
        scratch_types=[
            pltpu.VMEM((4, 2, CHUNK), jnp.int32),
            [pltpu.VMEM((CHUNK, D), jnp.float32) for _ in range(2)],
            [pltpu.SemaphoreType.DMA for _ in range(4)],
            [pltpu.SemaphoreType.DMA for _ in range(2)],
            [pltpu.SemaphoreType.DMA for _ in range(2)],
            pltpu.VMEM_SHARED((NPAD, D), jnp.float32),
        ],
    )
    return kern(xs, ei2)


_BLK = 1280
_NBLK = NPAD // _BLK


def _dinv_of(dega_blk, degb_blk):
    deg = dega_blk[:, 0:1] + degb_blk[:, 0:1] + 1.0
    return lax.rsqrt(deg)


def _tc1_body(x_ref, w_ref, dega_ref, degb_ref, xs_ref):
    xw = jnp.dot(x_ref[...], w_ref[...], preferred_element_type=jnp.float32)
    xs_ref[...] = _dinv_of(dega_ref[...], degb_ref[...]) * xw


@jax.jit
def _tc_xs1(x_p, W1, dega, degb):
    return pl.pallas_call(
        _tc1_body,
        grid=(_NBLK,),
        in_specs=[
            pl.BlockSpec((_BLK, D), lambda i: (i, 0)),
            pl.BlockSpec((D, D), lambda i: (0, 0)),
            pl.BlockSpec((_BLK, 16), lambda i: (i, 0)),
            pl.BlockSpec((_BLK, 16), lambda i: (i, 0)),
        ],
        out_specs=pl.BlockSpec((_BLK, D), lambda i: (i, 0)),
        out_shape=jax.ShapeDtypeStruct((NPAD, D), jnp.float32),
    )(x_p, W1, dega, degb)


def _tc2_body(agga_ref, aggb_ref, xs_ref, dega_ref, degb_ref, b_ref, w_ref,
              out_ref):
    dinv = _dinv_of(dega_ref[...], degb_ref[...])
    pre = dinv * (agga_ref[...] + aggb_ref[...] + xs_ref[...]) + b_ref[...]
    h = jnp.maximum(pre, 0.0)
    hw = jnp.dot(h, w_ref[...], preferred_element_type=jnp.float32)
    out_ref[...] = dinv * hw


@jax.jit
def _tc_xs2(agga, aggb, xs1, dega, degb, b1, W2):
    return pl.pallas_call(
        _tc2_body,
        grid=(_NBLK,),
        in_specs=[
            pl.BlockSpec((_BLK, D), lambda i: (i, 0)),
            pl.BlockSpec((_BLK, D), lambda i: (i, 0)),
            pl.BlockSpec((_BLK, D), lambda i: (i, 0)),
            pl.BlockSpec((_BLK, 16), lambda i: (i, 0)),
            pl.BlockSpec((_BLK, 16), lambda i: (i, 0)),
            pl.BlockSpec((1, D), lambda i: (0, 0)),
            pl.BlockSpec((D, D), lambda i: (0, 0)),
        ],
        out_specs=pl.BlockSpec((_BLK, D), lambda i: (i, 0)),
        out_shape=jax.ShapeDtypeStruct((NPAD, D), jnp.float32),
    )(agga, aggb, xs1, dega, degb, b1, W2)


def _tc3_body(agga_ref, aggb_ref, xs_ref, dega_ref, degb_ref, b_ref,
              batch_ref, wfc_ref, bfc_ref, out_ref, sums_ref, counts_ref):
    i = pl.program_id(0)

    @pl.when(i == 0)
    def _():
        sums_ref[...] = jnp.zeros_like(sums_ref)
        counts_ref[...] = jnp.zeros_like(counts_ref)

    dinv = _dinv_of(dega_ref[...], degb_ref[...])
    pre = dinv * (agga_ref[...] + aggb_ref[...] + xs_ref[...]) + b_ref[...]
    h = jnp.maximum(pre, 0.0)

    gids = lax.broadcasted_iota(jnp.int32, (_BLK, N_GRAPHS), 1)
    onehot = (batch_ref[...] == gids).astype(jnp.float32)
    dn = (((0,), (0,)), ((), ()))
    sums_ref[...] += lax.dot_general(onehot, h, dn,
                                     preferred_element_type=jnp.float32)
    counts_ref[...] += lax.dot_general(
        onehot, jnp.ones((_BLK, D), jnp.float32), dn,
        preferred_element_type=jnp.float32)

    @pl.when(i == _NBLK - 1)
    def _():
        pooled = sums_ref[...] / jnp.maximum(counts_ref[...], 1.0)
        o = jnp.dot(pooled, wfc_ref[...], preferred_element_type=jnp.float32)
        out_ref[...] = jnp.maximum(o + bfc_ref[...], 0.0)


@jax.jit
def _tc_pool(agga, aggb, xs2, dega, degb, b2, batch_col, Wfc, bfc):
    return pl.pallas_call(
        _tc3_body,
        grid=(_NBLK,),
        in_specs=[
            pl.BlockSpec((_BLK, D), lambda i: (i, 0)),
            pl.BlockSpec((_BLK, D), lambda i: (i, 0)),
            pl.BlockSpec((_BLK, D), lambda i: (i, 0)),
            pl.BlockSpec((_BLK, 16), lambda i: (i, 0)),
            pl.BlockSpec((_BLK, 16), lambda i: (i, 0)),
            pl.BlockSpec((1, D), lambda i: (0, 0)),
            pl.BlockSpec((_BLK, 1), lambda i: (i, 0)),
            pl.BlockSpec((D, D), lambda i: (0, 0)),
            pl.BlockSpec((1, D), lambda i: (0, 0)),
        ],
        out_specs=pl.BlockSpec((N_GRAPHS, D), lambda i: (0, 0)),
        out_shape=jax.ShapeDtypeStruct((N_GRAPHS, D), jnp.float32),
        scratch_shapes=[
            pltpu.VMEM((N_GRAPHS, D), jnp.float32),
            pltpu.VMEM((N_GRAPHS, D), jnp.float32),
        ],
    )(agga, aggb, xs2, dega, degb, b2, batch_col, Wfc, bfc)


def kernel(x, edge_index, batch, W1, b1, W2, b2, Wfc, bfc):
    src = edge_index[0].astype(jnp.int32)
    dst = edge_index[1].astype(jnp.int32)
    fill = jnp.full((EPAD - N_EDGES,), PAD_ROW, jnp.int32)
    src_p = jnp.concatenate([src, fill])
    dst_p = jnp.concatenate([dst, fill])
    ei2 = jnp.stack([src_p.reshape(EPAD // CHUNK, CHUNK),
                     dst_p.reshape(EPAD // CHUNK, CHUNK)], axis=1)
    x_p = jnp.pad(x, ((0, NPAD - N_NODES), (0, 0)))
    batch_col = jnp.pad(batch.astype(jnp.int32), (0, NPAD - N_NODES),
                        constant_values=N_GRAPHS).reshape(NPAD, 1)
    b1r = b1.reshape(1, D)
    b2r = b2.reshape(1, D)
    bfcr = bfc.reshape(1, D)

    deg = _sc_degree(dst_p)
    dega, degb = deg[0], deg[1]
    xs1 = _tc_xs1(x_p, W1, dega, degb)
    agg1 = _sc_aggregate(xs1, ei2)
    xs2 = _tc_xs2(agg1[0], agg1[1], xs1, dega, degb, b1r, W2)
    agg2 = _sc_aggregate(xs2, ei2)
    return _tc_pool(agg2[0], agg2[1], xs2, dega, degb, b2r, batch_col, Wfc, bfcr)


# speedup vs baseline: 9.9584x; 1.0021x over previous
import functools

import jax
import jax.numpy as jnp
from jax import lax
from jax.experimental import pallas as pl
from jax.experimental.pallas import tpu as pltpu
from jax.experimental.pallas import tpu_sc as plsc

N_NODES = 10000
N_EDGES = 320000
D = 128
N_GRAPHS = 64

NC, NS = 2, 16
NW = NC * NS
CHUNK = 80
DCH = 128
NPAD = 10240
EPAD = 327680
EW = EPAD // NW
ROWS_W = NPAD // NS
PAD_ROW = N_NODES + 8

_SC_MESH = plsc.VectorSubcoreMesh(core_axis_name="c", subcore_axis_name="s")


def _deg_body(dst_hbm, out_hbm, dst_v, ones_v, zero_v, acc_sh):
    c = lax.axis_index("c")
    s = lax.axis_index("s")
    wid = s * NC + c

    @pl.loop(0, DCH)
    def _(i):
        ones_v[i, :] = jnp.ones((16,), jnp.float32)
        zero_v[i, :] = jnp.zeros((16,), jnp.float32)

    @pl.loop(0, ROWS_W, step=DCH)
    def _(j):
        pltpu.sync_copy(zero_v, acc_sh.at[pl.ds(s * ROWS_W + j, DCH)])

    plsc.subcore_barrier()

    @pl.loop(0, EW, step=DCH)
    def _(k):
        pltpu.sync_copy(dst_hbm.at[pl.ds(wid * EW + k, DCH)], dst_v)
        pltpu.sync_copy(ones_v, acc_sh.at[dst_v], add=True)

    plsc.subcore_barrier()
    pltpu.sync_copy(acc_sh.at[pl.ds(s * ROWS_W, ROWS_W)],
                    out_hbm.at[c].at[pl.ds(s * ROWS_W, ROWS_W)])


@jax.jit
def _sc_degree(dst_p):
    kern = pl.kernel(
        _deg_body,
        out_type=jax.ShapeDtypeStruct((NC, NPAD, 16), jnp.float32),
        mesh=_SC_MESH,
        scratch_types=[
            pltpu.VMEM((DCH,), jnp.int32),
            pltpu.VMEM((DCH, 16), jnp.float32),
            pltpu.VMEM((DCH, 16), jnp.float32),
            pltpu.VMEM_SHARED((NPAD, 16), jnp.float32),
        ],
    )
    return kern(dst_p)


NCH = EW // CHUNK


def _agg_body(xs_hbm, ei_hbm, out_hbm, idx_v, rows, isems, gsems, ssems,
              acc_sh):
    c = lax.axis_index("c")
    s = lax.axis_index("s")

    r0 = rows[0]

    @pl.loop(0, CHUNK)
    def _(i):
        @pl.loop(0, D, step=16)
        def _(j):
            r0[i, pl.ds(j, 16)] = jnp.zeros((16,), jnp.float32)

    @pl.loop(0, ROWS_W, step=CHUNK)
    def _(j):
        pltpu.sync_copy(r0, acc_sh.at[pl.ds(s * ROWS_W + j, CHUNK)])

    plsc.subcore_barrier()

    gbase = (s * NC + c) * NCH
    t4 = NCH // 4

    def load_idx(ch, q):
        pltpu.async_copy(ei_hbm.at[gbase + ch], idx_v.at[q], isems[q])

    def wait_idx(q):
        pltpu.make_async_copy(ei_hbm.at[gbase], idx_v.at[q],
                              isems[q]).wait()

    def start_gather(q, r):
        pltpu.async_copy(xs_hbm.at[idx_v.at[q].at[0]], rows[r], gsems[r])

    def wait_gather(r):
        pltpu.make_async_copy(xs_hbm.at[idx_v.at[0].at[0]], rows[r],
                              gsems[r]).wait()

    def start_scatter(q, r):
        pltpu.async_copy(rows[r], acc_sh.at[idx_v.at[q].at[1]], ssems[r],
                         add=True)

    def wait_scatter(r):
        pltpu.make_async_copy(rows[r], acc_sh.at[idx_v.at[0].at[1]],
                              ssems[r]).wait()

    load_idx(0, 0)
    load_idx(1, 1)
    load_idx(2, 2)
    wait_idx(0)
    start_gather(0, 0)
    wait_idx(1)
    start_gather(1, 1)

    @pl.loop(0, t4)
    def _(t):
        for j in range(4):
            ch = t * 4 + j
            jp1, jp2, jp3 = (j + 1) % 4, (j + 2) % 4, (j + 3) % 4

            if j == 0:
                @pl.when(t > 0)
                def _():
                    wait_scatter(jp3)
            else:
                wait_scatter(jp3)

            if j == 0:
                load_idx(ch + 3, jp3)
            else:
                @pl.when(t < t4 - 1)
                def _():
                    load_idx(ch + 3, jp3)

            def step_g():
                wait_idx(jp2)
                start_gather(jp2, jp2)

            if j < 2:
                step_g()
            else:
                @pl.when(t < t4 - 1)
                def _():
                    step_g()

            wait_gather(j)
            start_scatter(j, j)

    wait_scatter(3)

    plsc.subcore_barrier()
    pltpu.sync_copy(acc_sh.at[pl.ds(s * ROWS_W, ROWS_W)],
                    out_hbm.at[c].at[pl.ds(s * ROWS_W, ROWS_W)])


@jax.jit
def _sc_aggregate(xs, ei2):
    kern = pl.kernel(
        _agg_body,
        out_type=jax.ShapeDtypeStruct((NC, NPAD, D), jnp.float32),
        mesh=_SC_MESH,
        scratch_types=[
            pltpu.VMEM((4, 2, CHUNK), jnp.int32),
            [pltpu.VMEM((CHUNK, D), jnp.float32) for _ in range(4)],
            [pltpu.SemaphoreType.DMA for _ in range(4)],
            [pltpu.SemaphoreType.DMA for _ in range(4)],
            [pltpu.SemaphoreType.DMA for _ in range(4)],
            pltpu.VMEM_SHARED((NPAD, D), jnp.float32),
        ],
    )
    return kern(xs, ei2)


_BLK = 1280
_NBLK = NPAD // _BLK


def _dinv_of(dega_blk, degb_blk):
    deg = dega_blk[:, 0:1] + degb_blk[:, 0:1] + 1.0
    return lax.rsqrt(deg)


def _tc1_body(x_ref, w_ref, dega_ref, degb_ref, xs_ref):
    xw = jnp.dot(x_ref[...], w_ref[...], preferred_element_type=jnp.float32)
    xs_ref[...] = _dinv_of(dega_ref[...], degb_ref[...]) * xw


@jax.jit
def _tc_xs1(x_p, W1, dega, degb):
    return pl.pallas_call(
        _tc1_body,
        grid=(_NBLK,),
        in_specs=[
            pl.BlockSpec((_BLK, D), lambda i: (i, 0)),
            pl.BlockSpec((D, D), lambda i: (0, 0)),
            pl.BlockSpec((_BLK, 16), lambda i: (i, 0)),
            pl.BlockSpec((_BLK, 16), lambda i: (i, 0)),
        ],
        out_specs=pl.BlockSpec((_BLK, D), lambda i: (i, 0)),
        out_shape=jax.ShapeDtypeStruct((NPAD, D), jnp.float32),
    )(x_p, W1, dega, degb)


def _tc2_body(agga_ref, aggb_ref, xs_ref, dega_ref, degb_ref, b_ref, w_ref,
              out_ref):
    dinv = _dinv_of(dega_ref[...], degb_ref[...])
    pre = dinv * (agga_ref[...] + aggb_ref[...] + xs_ref[...]) + b_ref[...]
    h = jnp.maximum(pre, 0.0)
    hw = jnp.dot(h, w_ref[...], preferred_element_type=jnp.float32)
    out_ref[...] = dinv * hw


@jax.jit
def _tc_xs2(agga, aggb, xs1, dega, degb, b1, W2):
    return pl.pallas_call(
        _tc2_body,
        grid=(_NBLK,),
        in_specs=[
            pl.BlockSpec((_BLK, D), lambda i: (i, 0)),
            pl.BlockSpec((_BLK, D), lambda i: (i, 0)),
            pl.BlockSpec((_BLK, D), lambda i: (i, 0)),
            pl.BlockSpec((_BLK, 16), lambda i: (i, 0)),
            pl.BlockSpec((_BLK, 16), lambda i: (i, 0)),
            pl.BlockSpec((1, D), lambda i: (0, 0)),
            pl.BlockSpec((D, D), lambda i: (0, 0)),
        ],
        out_specs=pl.BlockSpec((_BLK, D), lambda i: (i, 0)),
        out_shape=jax.ShapeDtypeStruct((NPAD, D), jnp.float32),
    )(agga, aggb, xs1, dega, degb, b1, W2)


def _tc3_body(agga_ref, aggb_ref, xs_ref, dega_ref, degb_ref, b_ref,
              batch_ref, wfc_ref, bfc_ref, out_ref, sums_ref, counts_ref):
    i = pl.program_id(0)

    @pl.when(i == 0)
    def _():
        sums_ref[...] = jnp.zeros_like(sums_ref)
        counts_ref[...] = jnp.zeros_like(counts_ref)

    dinv = _dinv_of(dega_ref[...], degb_ref[...])
    pre = dinv * (agga_ref[...] + aggb_ref[...] + xs_ref[...]) + b_ref[...]
    h = jnp.maximum(pre, 0.0)

    gids = lax.broadcasted_iota(jnp.int32, (_BLK, N_GRAPHS), 1)
    onehot = (batch_ref[...] == gids).astype(jnp.float32)
    dn = (((0,), (0,)), ((), ()))
    sums_ref[...] += lax.dot_general(onehot, h, dn,
                                     preferred_element_type=jnp.float32)
    counts_ref[...] += lax.dot_general(
        onehot, jnp.ones((_BLK, D), jnp.float32), dn,
        preferred_element_type=jnp.float32)

    @pl.when(i == _NBLK - 1)
    def _():
        pooled = sums_ref[...] / jnp.maximum(counts_ref[...], 1.0)
        o = jnp.dot(pooled, wfc_ref[...], preferred_element_type=jnp.float32)
        out_ref[...] = jnp.maximum(o + bfc_ref[...], 0.0)


@jax.jit
def _tc_pool(agga, aggb, xs2, dega, degb, b2, batch_col, Wfc, bfc):
    return pl.pallas_call(
        _tc3_body,
        grid=(_NBLK,),
        in_specs=[
            pl.BlockSpec((_BLK, D), lambda i: (i, 0)),
            pl.BlockSpec((_BLK, D), lambda i: (i, 0)),
            pl.BlockSpec((_BLK, D), lambda i: (i, 0)),
            pl.BlockSpec((_BLK, 16), lambda i: (i, 0)),
            pl.BlockSpec((_BLK, 16), lambda i: (i, 0)),
            pl.BlockSpec((1, D), lambda i: (0, 0)),
            pl.BlockSpec((_BLK, 1), lambda i: (i, 0)),
            pl.BlockSpec((D, D), lambda i: (0, 0)),
            pl.BlockSpec((1, D), lambda i: (0, 0)),
        ],
        out_specs=pl.BlockSpec((N_GRAPHS, D), lambda i: (0, 0)),
        out_shape=jax.ShapeDtypeStruct((N_GRAPHS, D), jnp.float32),
        scratch_shapes=[
            pltpu.VMEM((N_GRAPHS, D), jnp.float32),
            pltpu.VMEM((N_GRAPHS, D), jnp.float32),
        ],
    )(agga, aggb, xs2, dega, degb, b2, batch_col, Wfc, bfc)


def kernel(x, edge_index, batch, W1, b1, W2, b2, Wfc, bfc):
    src = edge_index[0].astype(jnp.int32)
    dst = edge_index[1].astype(jnp.int32)
    fill = jnp.full((EPAD - N_EDGES,), PAD_ROW, jnp.int32)
    src_p = jnp.concatenate([src, fill])
    dst_p = jnp.concatenate([dst, fill])
    ei2 = jnp.stack([src_p.reshape(EPAD // CHUNK, CHUNK),
                     dst_p.reshape(EPAD // CHUNK, CHUNK)], axis=1)
    x_p = jnp.pad(x, ((0, NPAD - N_NODES), (0, 0)))
    batch_col = jnp.pad(batch.astype(jnp.int32), (0, NPAD - N_NODES),
                        constant_values=N_GRAPHS).reshape(NPAD, 1)
    b1r = b1.reshape(1, D)
    b2r = b2.reshape(1, D)
    bfcr = bfc.reshape(1, D)

    deg = _sc_degree(dst_p)
    dega, degb = deg[0], deg[1]
    xs1 = _tc_xs1(x_p, W1, dega, degb)
    agg1 = _sc_aggregate(xs1, ei2)
    xs2 = _tc_xs2(agg1[0], agg1[1], xs1, dega, degb, b1r, W2)
    agg2 = _sc_aggregate(xs2, ei2)
    return _tc_pool(agg2[0], agg2[1], xs2, dega, degb, b2r, batch_col, Wfc, bfcr)
